# baseline (speedup 1.0000x reference)
import functools

import jax
import jax.numpy as jnp
from jax import lax
from jax.experimental import pallas as pl
from jax.experimental.pallas import tpu as pltpu
from jax.experimental.pallas import tpu_sc as plsc

D_MODEL = 64
SCALE = 8.0
LANES = 16

NUM_CORES = 2
NUM_SUBCORES = 16
NUM_WORKERS = NUM_CORES * NUM_SUBCORES

B_TOTAL = 4096 * 50
B_PER_WORKER = B_TOTAL // NUM_WORKERS
CHUNK = 400
N_CHUNKS = B_PER_WORKER // CHUNK


def _make_gather_kernel():
    mesh = plsc.VectorSubcoreMesh(core_axis_name="c", subcore_axis_name="s")

    @functools.partial(
        pl.kernel,
        mesh=mesh,
        out_type=jax.ShapeDtypeStruct((B_TOTAL, D_MODEL), jnp.float32),
        scratch_types=[
            pltpu.VMEM((CHUNK,), jnp.int32),
            pltpu.VMEM((CHUNK, D_MODEL), jnp.float32),
            pltpu.SemaphoreType.DMA,
        ],
    )
    def gather_scale(table_hbm, idx_hbm, out_hbm, idx_v, rows_v, sem):
        wid = lax.axis_index("s") * NUM_CORES + lax.axis_index("c")
        base = wid * B_PER_WORKER

        @pl.loop(0, N_CHUNKS)
        def _(c):
            off = base + c * CHUNK
            pltpu.sync_copy(idx_hbm.at[pl.ds(off, CHUNK)], idx_v)

            @pl.loop(0, CHUNK, step=LANES)
            def _(r):
                v = idx_v[pl.ds(r, LANES)]
                for j in range(LANES):
                    pltpu.async_copy(
                        table_hbm.at[pl.ds(v[j], 1), :],
                        rows_v.at[pl.ds(r + j, 1), :],
                        sem,
                    )

            @pl.loop(0, CHUNK)
            def _(r):
                pltpu.make_async_copy(
                    table_hbm.at[pl.ds(0, 1), :],
                    rows_v.at[pl.ds(0, 1), :],
                    sem,
                ).wait()

            @pl.loop(0, CHUNK)
            def _(r):
                @pl.loop(0, D_MODEL, step=LANES)
                def _(l):
                    slc = (pl.ds(r, 1), pl.ds(l, LANES))
                    rows_v.at[*slc][...] = rows_v.at[*slc][...] * SCALE

            pltpu.sync_copy(rows_v, out_hbm.at[pl.ds(off, CHUNK)])

    return gather_scale


_gather_scale = _make_gather_kernel()


@jax.jit
def kernel(x, table):
    idx = x.reshape(-1).astype(jnp.int32)
    out = _gather_scale(table, idx)
    return out.reshape(x.shape + (D_MODEL,))
